# baseline (speedup 1.0000x reference)
import functools

import jax
import jax.numpy as jnp
from jax import lax
from jax.experimental import pallas as pl
from jax.experimental.pallas import tpu as pltpu
from jax.experimental.pallas import tpu_sc as plsc

NC = 2
NS = 16
C = 128


def _mm1_body(x_ref, w_ref, lo_ref, hi_ref):
    m = jnp.dot(x_ref[...], w_ref[...], preferred_element_type=jnp.float32)
    h = m.shape[1] // 2
    lo_ref[...] = m[:, :h]
    hi_ref[...] = m[:, h:]


def _mm1(x, W1):
    N, K = x.shape
    H = W1.shape[1]
    BM = 1000
    return pl.pallas_call(
        _mm1_body,
        grid=(N // BM,),
        in_specs=[
            pl.BlockSpec((BM, K), lambda i: (i, 0)),
            pl.BlockSpec((K, H), lambda i: (0, 0)),
        ],
        out_specs=[
            pl.BlockSpec((BM, H // 2), lambda i: (i, 0)),
            pl.BlockSpec((BM, H // 2), lambda i: (i, 0)),
        ],
        out_shape=[jax.ShapeDtypeStruct((N, H // 2), jnp.float32)] * 2,
    )(x, W1)


def _mm2_body(alo_ref, ahi_ref, w_ref, b_ref, o_ref):
    K = w_ref.shape[0]
    h = K // 2
    a0 = alo_ref[...] + b_ref[:, :h]
    a1 = ahi_ref[...] + b_ref[:, h:]
    m = jnp.dot(a0, w_ref[:h, :], preferred_element_type=jnp.float32)
    o_ref[...] = m + jnp.dot(a1, w_ref[h:, :], preferred_element_type=jnp.float32)


def _mm2(a_lo, a_hi, W2, b1_row):
    N, h = a_lo.shape
    K, OUT = W2.shape
    BM = 1000
    return pl.pallas_call(
        _mm2_body,
        grid=(N // BM,),
        in_specs=[
            pl.BlockSpec((BM, h), lambda i: (i, 0)),
            pl.BlockSpec((BM, h), lambda i: (i, 0)),
            pl.BlockSpec((K, OUT), lambda i: (0, 0)),
            pl.BlockSpec((1, K), lambda i: (0, 0)),
        ],
        out_specs=pl.BlockSpec((BM, OUT), lambda i: (i, 0)),
        out_shape=jax.ShapeDtypeStruct((N, OUT), jnp.float32),
    )(a_lo, a_hi, W2, b1_row)


def _softmax_body(p0_ref, p1_ref, b_ref, o_ref):
    z = p0_ref[...] + p1_ref[...] + b_ref[...]
    z = z - jnp.max(z, axis=1, keepdims=True)
    e = jnp.exp(z)
    o_ref[...] = e / jnp.sum(e, axis=1, keepdims=True)


def _softmax(p0, p1, b2_row):
    N, OUT = p0.shape
    BM = 1000
    return pl.pallas_call(
        _softmax_body,
        grid=(N // BM,),
        in_specs=[
            pl.BlockSpec((BM, OUT), lambda i: (i, 0)),
            pl.BlockSpec((BM, OUT), lambda i: (i, 0)),
            pl.BlockSpec((1, OUT), lambda i: (0, 0)),
        ],
        out_specs=pl.BlockSpec((BM, OUT), lambda i: (i, 0)),
        out_shape=jax.ShapeDtypeStruct((N, OUT), jnp.float32),
    )(p0, p1, b2_row)


def _acc_rows(N):
    return ((N + 1 + NS * 8 - 1) // (NS * 8)) * (NS * 8)


NB = 16


def _pipelined_chunks(m_hbm, acc, src2, dst2, row_base,
                      sidxs, didxs, rows, sems, CH):
    def batch(bi, carry):
        boff = pl.multiple_of(row_base + bi * NB, 8)
        pltpu.sync_copy(src2.at[pl.ds(boff, NB)], sidxs)
        pltpu.sync_copy(dst2.at[pl.ds(boff, NB)], didxs)
        pltpu.async_copy(m_hbm.at[sidxs.at[0]], rows[0], sems[0])
        for j in range(NB):
            b = j % 2
            pltpu.make_async_copy(m_hbm.at[sidxs.at[0]], rows[b],
                                  sems[b]).wait()
            if j + 1 < NB:
                pltpu.async_copy(m_hbm.at[sidxs.at[j + 1]], rows[b ^ 1],
                                 sems[b ^ 1])
            pltpu.sync_copy(rows[b], acc.at[didxs.at[j]], add=True)
        return carry

    lax.fori_loop(0, CH // NB, batch, 0)


def _make_scatter(N, W, E_pad):
    CH = E_pad // (NS * C)
    assert CH % NB == 0
    npad = _acc_rows(N)
    zrows = npad // NS
    mesh = plsc.VectorSubcoreMesh(core_axis_name="c", subcore_axis_name="s")

    @functools.partial(
        pl.kernel,
        out_type=[jax.ShapeDtypeStruct((npad, W), jnp.float32)] * 2,
        mesh=mesh,
        scratch_types=[
            pltpu.VMEM((NB, C), jnp.int32),
            pltpu.VMEM((NB, C), jnp.int32),
            pltpu.VMEM((C, W), jnp.float32),
            pltpu.VMEM((C, W), jnp.float32),
            pltpu.VMEM_SHARED((npad, W), jnp.float32),
            pltpu.SemaphoreType.DMA,
            pltpu.SemaphoreType.DMA,
        ],
    )
    def scat(mlo, mhi, src2, dst2, zz, olo, ohi,
             sidxs, didxs, rows0, rows1, acc, sem0, sem1):
        cid = lax.axis_index("c")
        t = lax.axis_index("s")
        soff = pl.multiple_of(t * zrows, 8)

        pltpu.sync_copy(zz.at[pl.ds(soff, zrows)], acc.at[pl.ds(soff, zrows)])
        plsc.subcore_barrier()

        def run(m_hbm, out_hbm):
            _pipelined_chunks(m_hbm, acc, src2, dst2, t * CH,
                              sidxs, didxs, [rows0, rows1],
                              [sem0, sem1], CH)
            plsc.subcore_barrier()
            pltpu.sync_copy(acc.at[pl.ds(soff, zrows)],
                            out_hbm.at[pl.ds(soff, zrows)])

        @pl.when(cid == 0)
        def _():
            run(mlo, olo)

        @pl.when(cid != 0)
        def _():
            run(mhi, ohi)

    return scat


def _make_scatter_esplit(N, W, E_pad):
    CH = E_pad // (NC * NS * C)
    assert CH % NB == 0
    npad = _acc_rows(N)
    zrows = npad // NS
    mesh = plsc.VectorSubcoreMesh(core_axis_name="c", subcore_axis_name="s")

    @functools.partial(
        pl.kernel,
        out_type=[jax.ShapeDtypeStruct((npad, W), jnp.float32)] * 2,
        mesh=mesh,
        scratch_types=[
            pltpu.VMEM((NB, C), jnp.int32),
            pltpu.VMEM((NB, C), jnp.int32),
            pltpu.VMEM((C, W), jnp.float32),
            pltpu.VMEM((C, W), jnp.float32),
            pltpu.VMEM_SHARED((npad, W), jnp.float32),
            pltpu.SemaphoreType.DMA,
            pltpu.SemaphoreType.DMA,
        ],
    )
    def scat(m, src2, dst2, zz, o0, o1,
             sidxs, didxs, rows0, rows1, acc, sem0, sem1):
        cid = lax.axis_index("c")
        t = lax.axis_index("s")
        soff = pl.multiple_of(t * zrows, 8)

        pltpu.sync_copy(zz.at[pl.ds(soff, zrows)], acc.at[pl.ds(soff, zrows)])
        plsc.subcore_barrier()

        _pipelined_chunks(m, acc, src2, dst2, (cid * NS + t) * CH,
                          sidxs, didxs, [rows0, rows1], [sem0, sem1], CH)
        plsc.subcore_barrier()

        @pl.when(cid == 0)
        def _():
            pltpu.sync_copy(acc.at[pl.ds(soff, zrows)],
                            o0.at[pl.ds(soff, zrows)])

        @pl.when(cid != 0)
        def _():
            pltpu.sync_copy(acc.at[pl.ds(soff, zrows)],
                            o1.at[pl.ds(soff, zrows)])

    return scat


def kernel(x, edge_index, W1, b1, W2, b2):
    N, IN = x.shape
    H = W1.shape[1]
    OUT = W2.shape[1]
    E = edge_index.shape[1]

    src = edge_index[0].astype(jnp.int32)
    dst = edge_index[1].astype(jnp.int32)

    quantum = NC * NS * C * 8
    E_pad = -(-E // quantum) * quantum
    pad = E_pad - E
    npad = _acc_rows(N)
    trash = N + jnp.arange(pad, dtype=jnp.int32) % (npad - N)
    src_p = jnp.concatenate([src, jnp.zeros((pad,), jnp.int32)]).reshape(-1, C)
    dst_p = jnp.concatenate([dst, trash]).reshape(-1, C)
    z1 = jnp.zeros((npad, H // 2), jnp.float32)
    z2 = jnp.zeros((npad, OUT), jnp.float32)

    m1lo, m1hi = _mm1(x, W1)
    a1lo, a1hi = _make_scatter(N, H // 2, E_pad)(m1lo, m1hi, src_p, dst_p, z1)
    m2 = _mm2(a1lo[:N], a1hi[:N], W2, b1.reshape(1, -1))
    p0, p1 = _make_scatter_esplit(N, OUT, E_pad)(m2, src_p, dst_p, z2)
    return _softmax(p0[:N], p1[:N], b2.reshape(1, -1))


# speedup vs baseline: 2.6126x; 2.6126x over previous
import functools

import jax
import jax.numpy as jnp
from jax import lax
from jax.experimental import pallas as pl
from jax.experimental.pallas import tpu as pltpu
from jax.experimental.pallas import tpu_sc as plsc

NC = 2
NS = 16
C = 128


def _mm1_body(x_ref, w_ref, lo_ref, hi_ref):
    m = jnp.dot(x_ref[...], w_ref[...], preferred_element_type=jnp.float32)
    h = m.shape[1] // 2
    lo_ref[...] = m[:, :h]
    hi_ref[...] = m[:, h:]


def _mm1(x, W1):
    N, K = x.shape
    H = W1.shape[1]
    BM = 1000
    return pl.pallas_call(
        _mm1_body,
        grid=(N // BM,),
        in_specs=[
            pl.BlockSpec((BM, K), lambda i: (i, 0)),
            pl.BlockSpec((K, H), lambda i: (0, 0)),
        ],
        out_specs=[
            pl.BlockSpec((BM, H // 2), lambda i: (i, 0)),
            pl.BlockSpec((BM, H // 2), lambda i: (i, 0)),
        ],
        out_shape=[jax.ShapeDtypeStruct((N, H // 2), jnp.float32)] * 2,
    )(x, W1)


def _mm2_body(alo_ref, ahi_ref, w_ref, b_ref, o_ref):
    K = w_ref.shape[0]
    h = K // 2
    a0 = alo_ref[...] + b_ref[:, :h]
    a1 = ahi_ref[...] + b_ref[:, h:]
    m = jnp.dot(a0, w_ref[:h, :], preferred_element_type=jnp.float32)
    o_ref[...] = m + jnp.dot(a1, w_ref[h:, :], preferred_element_type=jnp.float32)


def _mm2(a_lo, a_hi, W2, b1_row):
    N, h = a_lo.shape
    K, OUT = W2.shape
    BM = 1000
    return pl.pallas_call(
        _mm2_body,
        grid=(N // BM,),
        in_specs=[
            pl.BlockSpec((BM, h), lambda i: (i, 0)),
            pl.BlockSpec((BM, h), lambda i: (i, 0)),
            pl.BlockSpec((K, OUT), lambda i: (0, 0)),
            pl.BlockSpec((1, K), lambda i: (0, 0)),
        ],
        out_specs=pl.BlockSpec((BM, OUT), lambda i: (i, 0)),
        out_shape=jax.ShapeDtypeStruct((N, OUT), jnp.float32),
    )(a_lo, a_hi, W2, b1_row)


def _softmax_body(p0_ref, p1_ref, b_ref, o_ref):
    z = p0_ref[...] + p1_ref[...] + b_ref[...]
    z = z - jnp.max(z, axis=1, keepdims=True)
    e = jnp.exp(z)
    o_ref[...] = e / jnp.sum(e, axis=1, keepdims=True)


def _softmax(p0, p1, b2_row):
    N, OUT = p0.shape
    BM = 1000
    return pl.pallas_call(
        _softmax_body,
        grid=(N // BM,),
        in_specs=[
            pl.BlockSpec((BM, OUT), lambda i: (i, 0)),
            pl.BlockSpec((BM, OUT), lambda i: (i, 0)),
            pl.BlockSpec((1, OUT), lambda i: (0, 0)),
        ],
        out_specs=pl.BlockSpec((BM, OUT), lambda i: (i, 0)),
        out_shape=jax.ShapeDtypeStruct((N, OUT), jnp.float32),
    )(p0, p1, b2_row)


def _acc_rows(N):
    return ((N + C + NS * 8 - 1) // (NS * 8)) * (NS * 8)


NB = 16


def _pipelined_chunks(m_hbm, acc, src2, dst2, row_base,
                      sidxs, didxs, rows, sems, CH):
    def batch(bi, carry):
        boff = pl.multiple_of(row_base + bi * NB, 8)
        pltpu.sync_copy(src2.at[pl.ds(boff, NB)], sidxs)
        pltpu.sync_copy(dst2.at[pl.ds(boff, NB)], didxs)
        pltpu.async_copy(m_hbm.at[sidxs.at[0]], rows[0], sems[0])
        for j in range(NB):
            b = j % 2
            pltpu.make_async_copy(m_hbm.at[sidxs.at[0]], rows[b],
                                  sems[b]).wait()
            if j + 1 < NB:
                pltpu.async_copy(m_hbm.at[sidxs.at[j + 1]], rows[b ^ 1],
                                 sems[b ^ 1])
            pltpu.sync_copy(rows[b], acc.at[didxs.at[j]], add=True)
        return carry

    lax.fori_loop(0, CH // NB, batch, 0)


def _make_scatter(N, W, E_pad):
    CH = E_pad // (NS * C)
    assert CH % NB == 0
    npad = _acc_rows(N)
    zrows = npad // NS
    mesh = plsc.VectorSubcoreMesh(core_axis_name="c", subcore_axis_name="s")

    @functools.partial(
        pl.kernel,
        out_type=[jax.ShapeDtypeStruct((npad, W), jnp.float32)] * 2,
        mesh=mesh,
        scratch_types=[
            pltpu.VMEM((NB, C), jnp.int32),
            pltpu.VMEM((NB, C), jnp.int32),
            pltpu.VMEM((C, W), jnp.float32),
            pltpu.VMEM((C, W), jnp.float32),
            pltpu.VMEM_SHARED((npad, W), jnp.float32),
            pltpu.SemaphoreType.DMA,
            pltpu.SemaphoreType.DMA,
        ],
    )
    def scat(mlo, mhi, src2, dst2, zz, olo, ohi,
             sidxs, didxs, rows0, rows1, acc, sem0, sem1):
        cid = lax.axis_index("c")
        t = lax.axis_index("s")
        soff = pl.multiple_of(t * zrows, 8)

        pltpu.sync_copy(zz.at[pl.ds(soff, zrows)], acc.at[pl.ds(soff, zrows)])
        plsc.subcore_barrier()

        def run(m_hbm, out_hbm):
            _pipelined_chunks(m_hbm, acc, src2, dst2, t * CH,
                              sidxs, didxs, [rows0, rows1],
                              [sem0, sem1], CH)
            plsc.subcore_barrier()
            pltpu.sync_copy(acc.at[pl.ds(soff, zrows)],
                            out_hbm.at[pl.ds(soff, zrows)])

        @pl.when(cid == 0)
        def _():
            run(mlo, olo)

        @pl.when(cid != 0)
        def _():
            run(mhi, ohi)

    return scat


def _make_scatter_esplit(N, W, E_pad):
    CH = E_pad // (NC * NS * C)
    assert CH % NB == 0
    npad = _acc_rows(N)
    zrows = npad // NS
    mesh = plsc.VectorSubcoreMesh(core_axis_name="c", subcore_axis_name="s")

    @functools.partial(
        pl.kernel,
        out_type=[jax.ShapeDtypeStruct((npad, W), jnp.float32)] * 2,
        mesh=mesh,
        scratch_types=[
            pltpu.VMEM((NB, C), jnp.int32),
            pltpu.VMEM((NB, C), jnp.int32),
            pltpu.VMEM((C, W), jnp.float32),
            pltpu.VMEM((C, W), jnp.float32),
            pltpu.VMEM_SHARED((npad, W), jnp.float32),
            pltpu.SemaphoreType.DMA,
            pltpu.SemaphoreType.DMA,
        ],
    )
    def scat(m, src2, dst2, zz, o0, o1,
             sidxs, didxs, rows0, rows1, acc, sem0, sem1):
        cid = lax.axis_index("c")
        t = lax.axis_index("s")
        soff = pl.multiple_of(t * zrows, 8)

        pltpu.sync_copy(zz.at[pl.ds(soff, zrows)], acc.at[pl.ds(soff, zrows)])
        plsc.subcore_barrier()

        _pipelined_chunks(m, acc, src2, dst2, (cid * NS + t) * CH,
                          sidxs, didxs, [rows0, rows1], [sem0, sem1], CH)
        plsc.subcore_barrier()

        @pl.when(cid == 0)
        def _():
            pltpu.sync_copy(acc.at[pl.ds(soff, zrows)],
                            o0.at[pl.ds(soff, zrows)])

        @pl.when(cid != 0)
        def _():
            pltpu.sync_copy(acc.at[pl.ds(soff, zrows)],
                            o1.at[pl.ds(soff, zrows)])

    return scat


def kernel(x, edge_index, W1, b1, W2, b2):
    N, IN = x.shape
    H = W1.shape[1]
    OUT = W2.shape[1]
    E = edge_index.shape[1]

    src = edge_index[0].astype(jnp.int32)
    dst = edge_index[1].astype(jnp.int32)

    quantum = NC * NS * C * 8
    E_pad = -(-E // quantum) * quantum
    pad = E_pad - E
    npad = _acc_rows(N)
    ar = jnp.arange(pad, dtype=jnp.int32)
    src_p = jnp.concatenate([src, ar % N]).reshape(-1, C)
    dst_p = jnp.concatenate([dst, N + ar % C]).reshape(-1, C)
    z1 = jnp.zeros((npad, H // 2), jnp.float32)
    z2 = jnp.zeros((npad, OUT), jnp.float32)

    m1lo, m1hi = _mm1(x, W1)
    a1lo, a1hi = _make_scatter(N, H // 2, E_pad)(m1lo, m1hi, src_p, dst_p, z1)
    m2 = _mm2(a1lo[:N], a1hi[:N], W2, b1.reshape(1, -1))
    p0, p1 = _make_scatter_esplit(N, OUT, E_pad)(m2, src_p, dst_p, z2)
    return _softmax(p0[:N], p1[:N], b2.reshape(1, -1))
